# baseline (speedup 1.0000x reference)
import jax
import jax.numpy as jnp
from jax import lax
from jax.experimental import pallas as pl
from jax.experimental.pallas import tpu as pltpu

IMGSZ = 640.0
REG_MAX = 16
NC = 80
CONF = 0.25
MAX_DETS = 300
CONF_BITS = 0x3E800000
ONE_BITS = 0x3F800000

SCALES = (
    (80, 80, 8.0, 6400, 0),
    (40, 40, 16.0, 1664, 6400),
    (20, 20, 32.0, 512, 8064),
)
AP = 8576
ROWS = AP // 128
P = 304
NV = 12


def _dot11(a, b):
    return lax.dot_general(a, b, (((1,), (1,)), ((), ())),
                           preferred_element_type=jnp.float32)


def _excl_cumsum(mask_f32):
    c = lax.broadcasted_iota(jnp.int32, (128, 128), 0)
    cc = lax.broadcasted_iota(jnp.int32, (128, 128), 1)
    m1 = (c < cc).astype(jnp.bfloat16)
    mb = mask_f32.astype(jnp.bfloat16)
    within = lax.dot_general(mb, m1, (((1,), (0,)), ((), ())),
                             preferred_element_type=jnp.float32)
    rowsum = jnp.sum(mask_f32, axis=1, keepdims=True)
    r = lax.broadcasted_iota(jnp.int32, (ROWS, ROWS), 0)
    rr = lax.broadcasted_iota(jnp.int32, (ROWS, ROWS), 1)
    m2 = (rr < r).astype(jnp.bfloat16)
    rowoff = lax.dot_general(m2, rowsum.astype(jnp.bfloat16),
                             (((1,), (0,)), ((), ())),
                             preferred_element_type=jnp.float32)
    return within + rowoff


def _body(p8_ref, p16_ref, p32_ref, det_ref, num_ref):
    keys = []
    vals = []
    for ref, (h, w, stride, hwpad, off) in zip((p8_ref, p16_ref, p32_ref),
                                               SCALES):
        x = ref[0]
        hw = h * w
        dists = []
        for s in range(4):
            sub = x[16 * s:16 * (s + 1), :]
            m = jnp.max(sub, axis=0, keepdims=True)
            e = jnp.exp(sub - m)
            den = jnp.sum(e, axis=0, keepdims=True)
            j = lax.broadcasted_iota(jnp.int32, (REG_MAX, hw), 0).astype(
                jnp.float32)
            num = jnp.sum(e * j, axis=0, keepdims=True)
            dists.append(num / den)
        sig = jax.nn.sigmoid(x[4 * REG_MAX:, :])
        bs = jnp.max(sig, axis=0, keepdims=True)
        rid = lax.broadcasted_iota(jnp.int32, (NC, hw), 0)
        bc = jnp.min(jnp.where(sig == bs, rid, NC + 47), axis=0,
                     keepdims=True).astype(jnp.float32)
        lane = lax.broadcasted_iota(jnp.int32, (1, hw), 1)
        ax = (lane % w).astype(jnp.float32) + 0.5
        ay = (lane // w).astype(jnp.float32) + 0.5
        x1 = jnp.clip((ax - dists[0]) * stride, 0.0, IMGSZ)
        y1 = jnp.clip((ay - dists[1]) * stride, 0.0, IMGSZ)
        x2 = jnp.clip((ax + dists[2]) * stride, 0.0, IMGSZ)
        y2 = jnp.clip((ay + dists[3]) * stride, 0.0, IMGSZ)
        kbits = lax.bitcast_convert_type(bs, jnp.int32)
        key = jnp.where(bs >= CONF, kbits, jnp.int32(-1))
        idx = lane + off
        chunks = [((key >> (8 * n)) & 255).astype(jnp.float32)
                  for n in range(4)]
        ich = [((idx >> (8 * n)) & 255).astype(jnp.float32)
               for n in range(2)]
        v = jnp.concatenate([x1, y1, x2, y2, bs, bc] + chunks + ich,
                            axis=0).astype(jnp.bfloat16)
        if hwpad > hw:
            pad = hwpad - hw
            v = jnp.pad(v, ((0, 0), (0, pad)))
            key = jnp.pad(key, ((0, 0), (0, pad)),
                          constant_values=-(2 ** 31))
        keys.append(key)
        vals.append(v)

    kflat = jnp.concatenate(keys, axis=1)
    vb = jnp.concatenate(vals, axis=1)
    kk = jnp.reshape(kflat, (ROWS, 128))

    lo0 = jnp.int32(CONF_BITS)
    hi0 = jnp.int32(ONE_BITS + 1)

    def step(_, lh):
        slo, shi = lh
        mid = (slo + shi) // 2
        cnt = jnp.sum((kk >= mid).astype(jnp.int32))
        ok = cnt >= MAX_DETS
        return (jnp.where(ok, mid, slo), jnp.where(ok, shi, mid))

    slo, _ = lax.fori_loop(0, 25, step, (lo0, hi0))
    cnt_pass = jnp.sum((kk >= lo0).astype(jnp.int32))
    t = jnp.where(cnt_pass >= MAX_DETS, slo, jnp.int32(-1))

    n_greater = jnp.sum((kk > t).astype(jnp.int32))
    need = (MAX_DETS - n_greater).astype(jnp.float32)
    tie = kk == t
    tcum = _excl_cumsum(tie.astype(jnp.float32))
    sel = (kk > t) | (tie & (tcum < need))
    sf = sel.astype(jnp.float32)
    pos = _excl_cumsum(sf)

    posf = jnp.reshape(pos, (1, AP))
    self_f = jnp.reshape(sf, (1, AP))
    slots = lax.broadcasted_iota(jnp.int32, (P, 1), 0).astype(jnp.float32)
    oh = ((slots == posf) & (self_f > 0.5)).astype(jnp.bfloat16)
    c = _dot11(oh, vb)
    ct = _dot11(vb, oh)

    def pair_of(c3, c2, c1, c0):
        hi = jnp.where(c3 >= 128.0, c3 - 256.0, c3) * 256.0 + c2
        lo = c1 * 256.0 + c0
        return hi, lo

    khp, klp = pair_of(c[:, 9:10], c[:, 8:9], c[:, 7:8], c[:, 6:7])
    khq, klq = pair_of(ct[9:10, :], ct[8:9, :], ct[7:8, :], ct[6:7, :])
    ip = c[:, 10:11] + 256.0 * c[:, 11:12]
    iq = ct[10:11, :] + 256.0 * ct[11:12, :]
    valid_p = lax.broadcasted_iota(jnp.int32, (P, 1), 0).astype(
        jnp.float32) < MAX_DETS
    valid_q = lax.broadcasted_iota(jnp.int32, (1, P), 1) < MAX_DETS
    beats = valid_p & (
        (khp > khq) | ((khp == khq) & ((klp > klq)
                                       | ((klp == klq) & (ip < iq)))))
    rank_q = jnp.sum(beats.astype(jnp.float32), axis=0, keepdims=True)

    out_r = lax.broadcasted_iota(jnp.int32, (P, 1), 0).astype(jnp.float32)
    oh2 = ((out_r == rank_q) & valid_q).astype(jnp.bfloat16)
    d = lax.dot_general(oh2, c[:, 0:6].astype(jnp.bfloat16),
                        (((1,), (0,)), ((), ())),
                        preferred_element_type=jnp.float32)
    det = d[0:MAX_DETS, :]
    score = jnp.maximum(det[:, 4:5], 0.0)
    det = jnp.concatenate([det[:, 0:4], score, det[:, 5:6]], axis=1)
    det_ref[...] = det[None]
    num_ref[...] = jnp.broadcast_to(jnp.minimum(cnt_pass, MAX_DETS),
                                    (1, 1, 128))


@jax.jit
def kernel(p8, p16, p32):
    b = p8.shape[0]
    r8 = p8.reshape(b, 144, 6400)
    r16 = p16.reshape(b, 144, 1600)
    r32 = p32.reshape(b, 144, 400)

    det, num = pl.pallas_call(
        _body,
        grid=(b,),
        in_specs=[
            pl.BlockSpec((1, 144, 6400), lambda i: (i, 0, 0)),
            pl.BlockSpec((1, 144, 1600), lambda i: (i, 0, 0)),
            pl.BlockSpec((1, 144, 400), lambda i: (i, 0, 0)),
        ],
        out_specs=[
            pl.BlockSpec((1, MAX_DETS, 6), lambda i: (i, 0, 0)),
            pl.BlockSpec((1, 1, 128), lambda i: (i, 0, 0)),
        ],
        out_shape=[
            jax.ShapeDtypeStruct((b, MAX_DETS, 6), jnp.float32),
            jax.ShapeDtypeStruct((b, 1, 128), jnp.int32),
        ],
        compiler_params=pltpu.CompilerParams(
            dimension_semantics=("arbitrary",)),
    )(r8, r16, r32)

    return det, num[:, 0, 0].astype(jnp.int64)


# speedup vs baseline: 1.1779x; 1.1779x over previous
import jax
import jax.numpy as jnp
from jax import lax
from jax.experimental import pallas as pl
from jax.experimental.pallas import tpu as pltpu

IMGSZ = 640.0
REG_MAX = 16
NC = 80
CONF = 0.25
MAX_DETS = 300
CONF_BITS = 0x3E800000
ONE_BITS = 0x3F800000

SCALES = (
    (80, 80, 8.0, 6400, 0),
    (40, 40, 16.0, 1664, 6400),
    (20, 20, 32.0, 512, 8064),
)
AP = 8576
ROWS = AP // 128
P = 304
NV = 12


def _dot11(a, b):
    return lax.dot_general(a, b, (((1,), (1,)), ((), ())),
                           preferred_element_type=jnp.float32)


def _excl_cumsum(mask_f32):
    c = lax.broadcasted_iota(jnp.int32, (128, 128), 0)
    cc = lax.broadcasted_iota(jnp.int32, (128, 128), 1)
    m1 = (c < cc).astype(jnp.bfloat16)
    mb = mask_f32.astype(jnp.bfloat16)
    within = lax.dot_general(mb, m1, (((1,), (0,)), ((), ())),
                             preferred_element_type=jnp.float32)
    rowsum = jnp.sum(mask_f32, axis=1, keepdims=True)
    r = lax.broadcasted_iota(jnp.int32, (ROWS, ROWS), 0)
    rr = lax.broadcasted_iota(jnp.int32, (ROWS, ROWS), 1)
    m2 = (rr < r).astype(jnp.bfloat16)
    rowoff = lax.dot_general(m2, rowsum.astype(jnp.bfloat16),
                             (((1,), (0,)), ((), ())),
                             preferred_element_type=jnp.float32)
    return within + rowoff


def _body(p8_ref, p16_ref, p32_ref, det_ref, num_ref):
    keys = []
    vals = []
    for ref, (h, w, stride, hwpad, off) in zip((p8_ref, p16_ref, p32_ref),
                                               SCALES):
        x = ref[0]
        hw = h * w
        dists = []
        for s in range(4):
            sub = x[16 * s:16 * (s + 1), :]
            m = jnp.max(sub, axis=0, keepdims=True)
            e = jnp.exp(sub - m)
            den = jnp.sum(e, axis=0, keepdims=True)
            j = lax.broadcasted_iota(jnp.int32, (REG_MAX, hw), 0).astype(
                jnp.float32)
            num = jnp.sum(e * j, axis=0, keepdims=True)
            dists.append(num / den)
        sig = jax.nn.sigmoid(x[4 * REG_MAX:, :])
        bs = jnp.max(sig, axis=0, keepdims=True)
        rid = lax.broadcasted_iota(jnp.int32, (NC, hw), 0)
        bc = jnp.min(jnp.where(sig == bs, rid, NC + 47), axis=0,
                     keepdims=True).astype(jnp.float32)
        lane = lax.broadcasted_iota(jnp.int32, (1, hw), 1)
        ax = (lane % w).astype(jnp.float32) + 0.5
        ay = (lane // w).astype(jnp.float32) + 0.5
        x1 = jnp.clip((ax - dists[0]) * stride, 0.0, IMGSZ)
        y1 = jnp.clip((ay - dists[1]) * stride, 0.0, IMGSZ)
        x2 = jnp.clip((ax + dists[2]) * stride, 0.0, IMGSZ)
        y2 = jnp.clip((ay + dists[3]) * stride, 0.0, IMGSZ)
        kbits = lax.bitcast_convert_type(bs, jnp.int32)
        key = jnp.where(bs >= CONF, kbits, jnp.int32(-1))
        idx = lane + off
        chunks = [((key >> (8 * n)) & 255).astype(jnp.float32)
                  for n in range(4)]
        ich = [((idx >> (8 * n)) & 255).astype(jnp.float32)
               for n in range(2)]
        v = jnp.concatenate([x1, y1, x2, y2, bs, bc] + chunks + ich,
                            axis=0).astype(jnp.bfloat16)
        if hwpad > hw:
            pad = hwpad - hw
            v = jnp.pad(v, ((0, 0), (0, pad)))
            key = jnp.pad(key, ((0, 0), (0, pad)),
                          constant_values=-(2 ** 31))
        keys.append(key)
        vals.append(v)

    kflat = jnp.concatenate(keys, axis=1)
    vb = jnp.concatenate(vals, axis=1)
    kk = jnp.reshape(kflat, (ROWS, 128))

    lo0 = jnp.int32(CONF_BITS)
    hi0 = jnp.int32(ONE_BITS + 1)

    def round16(_, lh):
        slo, shi = lh
        stp = jnp.maximum((shi - slo) // 16, 1)
        new_lo, new_hi = slo, shi
        for k in range(1, 16):
            m = slo + k * stp
            ok = jnp.sum((kk >= m).astype(jnp.int32)) >= MAX_DETS
            new_lo = jnp.where(ok, jnp.maximum(new_lo, m), new_lo)
            new_hi = jnp.where(ok, new_hi, jnp.minimum(new_hi, m))
        return (new_lo, new_hi)

    slo, _ = lax.fori_loop(0, 7, round16, (lo0, hi0))
    cnt_pass = jnp.sum((kk >= lo0).astype(jnp.int32))
    t = jnp.where(cnt_pass >= MAX_DETS, slo, jnp.int32(-1))

    n_greater = jnp.sum((kk > t).astype(jnp.int32))
    need = (MAX_DETS - n_greater).astype(jnp.float32)
    gt = kk > t
    tie = kk == t
    gcum = _excl_cumsum(gt.astype(jnp.float32))
    tcum = _excl_cumsum(tie.astype(jnp.float32))
    sel = gt | (tie & (tcum < need))
    pos = gcum + jnp.minimum(tcum, need)
    posx = jnp.where(sel, pos, 3.0e4)

    posf = jnp.reshape(posx, (1, AP))
    slots = lax.broadcasted_iota(jnp.int32, (P, 1), 0).astype(jnp.float32)
    oh = (slots == posf).astype(jnp.bfloat16)
    c = _dot11(oh, vb)
    ct = _dot11(vb, oh)

    def pair_of(c3, c2, c1, c0):
        hi = jnp.where(c3 >= 128.0, c3 - 256.0, c3) * 256.0 + c2
        lo = c1 * 256.0 + c0
        return hi, lo

    khp, klp = pair_of(c[:, 9:10], c[:, 8:9], c[:, 7:8], c[:, 6:7])
    khq, klq = pair_of(ct[9:10, :], ct[8:9, :], ct[7:8, :], ct[6:7, :])
    ip = c[:, 10:11] + 256.0 * c[:, 11:12]
    iq = ct[10:11, :] + 256.0 * ct[11:12, :]
    valid_p = lax.broadcasted_iota(jnp.int32, (P, 1), 0).astype(
        jnp.float32) < MAX_DETS
    valid_q = lax.broadcasted_iota(jnp.int32, (1, P), 1) < MAX_DETS
    beats = valid_p & (
        (khp > khq) | ((khp == khq) & ((klp > klq)
                                       | ((klp == klq) & (ip < iq)))))
    rank_q = jnp.sum(beats.astype(jnp.float32), axis=0, keepdims=True)

    out_r = lax.broadcasted_iota(jnp.int32, (P, 1), 0).astype(jnp.float32)
    oh2 = ((out_r == rank_q) & valid_q).astype(jnp.bfloat16)
    d = lax.dot_general(oh2, c[:, 0:6].astype(jnp.bfloat16),
                        (((1,), (0,)), ((), ())),
                        preferred_element_type=jnp.float32)
    det = d[0:MAX_DETS, :]
    score = jnp.maximum(det[:, 4:5], 0.0)
    det = jnp.concatenate([det[:, 0:4], score, det[:, 5:6]], axis=1)
    det_ref[...] = det[None]
    num_ref[...] = jnp.broadcast_to(jnp.minimum(cnt_pass, MAX_DETS),
                                    (1, 1, 128))


@jax.jit
def kernel(p8, p16, p32):
    b = p8.shape[0]
    r8 = p8.reshape(b, 144, 6400)
    r16 = p16.reshape(b, 144, 1600)
    r32 = p32.reshape(b, 144, 400)

    det, num = pl.pallas_call(
        _body,
        grid=(b,),
        in_specs=[
            pl.BlockSpec((1, 144, 6400), lambda i: (i, 0, 0)),
            pl.BlockSpec((1, 144, 1600), lambda i: (i, 0, 0)),
            pl.BlockSpec((1, 144, 400), lambda i: (i, 0, 0)),
        ],
        out_specs=[
            pl.BlockSpec((1, MAX_DETS, 6), lambda i: (i, 0, 0)),
            pl.BlockSpec((1, 1, 128), lambda i: (i, 0, 0)),
        ],
        out_shape=[
            jax.ShapeDtypeStruct((b, MAX_DETS, 6), jnp.float32),
            jax.ShapeDtypeStruct((b, 1, 128), jnp.int32),
        ],
        compiler_params=pltpu.CompilerParams(
            dimension_semantics=("arbitrary",)),
    )(r8, r16, r32)

    return det, num[:, 0, 0].astype(jnp.int64)


# speedup vs baseline: 1.1786x; 1.0006x over previous
import jax
import jax.numpy as jnp
from jax import lax
from jax.experimental import pallas as pl
from jax.experimental.pallas import tpu as pltpu

IMGSZ = 640.0
REG_MAX = 16
NC = 80
CONF = 0.25
MAX_DETS = 300
CONF_BITS = 0x3E800000
ONE_BITS = 0x3F800000

SCALES = (
    (80, 80, 8.0, 6400, 0),
    (40, 40, 16.0, 1664, 6400),
    (20, 20, 32.0, 512, 8064),
)
AP = 8576
ROWS = AP // 128
P = 304
NV = 12


def _dot11(a, b):
    return lax.dot_general(a, b, (((1,), (1,)), ((), ())),
                           preferred_element_type=jnp.float32)


def _excl_cumsum(mask_f32):
    c = lax.broadcasted_iota(jnp.int32, (128, 128), 0)
    cc = lax.broadcasted_iota(jnp.int32, (128, 128), 1)
    m1 = (c < cc).astype(jnp.bfloat16)
    mb = mask_f32.astype(jnp.bfloat16)
    within = lax.dot_general(mb, m1, (((1,), (0,)), ((), ())),
                             preferred_element_type=jnp.float32)
    rowsum = jnp.sum(mask_f32, axis=1, keepdims=True)
    r = lax.broadcasted_iota(jnp.int32, (ROWS, ROWS), 0)
    rr = lax.broadcasted_iota(jnp.int32, (ROWS, ROWS), 1)
    m2 = (rr < r).astype(jnp.bfloat16)
    rowoff = lax.dot_general(m2, rowsum.astype(jnp.bfloat16),
                             (((1,), (0,)), ((), ())),
                             preferred_element_type=jnp.float32)
    return within + rowoff


def _body(p8_ref, p16_ref, p32_ref, det_ref, num_ref):
    keys = []
    vals = []
    for ref, (h, w, stride, hwpad, off) in zip((p8_ref, p16_ref, p32_ref),
                                               SCALES):
        x = ref[0]
        hw = h * w
        dists = []
        for s in range(4):
            sub = x[16 * s:16 * (s + 1), :]
            m = jnp.max(sub, axis=0, keepdims=True)
            e = jnp.exp(sub - m)
            den = jnp.sum(e, axis=0, keepdims=True)
            j = lax.broadcasted_iota(jnp.int32, (REG_MAX, hw), 0).astype(
                jnp.float32)
            num = jnp.sum(e * j, axis=0, keepdims=True)
            dists.append(num / den)
        sig = jax.nn.sigmoid(x[4 * REG_MAX:, :])
        bs = jnp.max(sig, axis=0, keepdims=True)
        rid = lax.broadcasted_iota(jnp.int32, (NC, hw), 0)
        bc = jnp.min(jnp.where(sig == bs, rid, NC + 47), axis=0,
                     keepdims=True).astype(jnp.float32)
        lane = lax.broadcasted_iota(jnp.int32, (1, hw), 1)
        ax = (lane % w).astype(jnp.float32) + 0.5
        ay = (lane // w).astype(jnp.float32) + 0.5
        x1 = jnp.clip((ax - dists[0]) * stride, 0.0, IMGSZ)
        y1 = jnp.clip((ay - dists[1]) * stride, 0.0, IMGSZ)
        x2 = jnp.clip((ax + dists[2]) * stride, 0.0, IMGSZ)
        y2 = jnp.clip((ay + dists[3]) * stride, 0.0, IMGSZ)
        kbits = lax.bitcast_convert_type(bs, jnp.int32)
        key = jnp.where(bs >= CONF, kbits, jnp.int32(-1))
        idx = lane + off
        chunks = [((key >> (8 * n)) & 255).astype(jnp.float32)
                  for n in range(4)]
        ich = [((idx >> (8 * n)) & 255).astype(jnp.float32)
               for n in range(2)]
        v = jnp.concatenate([x1, y1, x2, y2, bs, bc] + chunks + ich,
                            axis=0).astype(jnp.bfloat16)
        if hwpad > hw:
            pad = hwpad - hw
            v = jnp.pad(v, ((0, 0), (0, pad)))
            key = jnp.pad(key, ((0, 0), (0, pad)),
                          constant_values=-(2 ** 31))
        keys.append(key)
        vals.append(v)

    kflat = jnp.concatenate(keys, axis=1)
    vb = jnp.concatenate(vals, axis=1)
    kk = jnp.reshape(kflat, (ROWS, 128))

    lo0 = jnp.int32(CONF_BITS)
    hi0 = jnp.int32(ONE_BITS + 1)

    def round16(_, lh):
        slo, shi = lh
        stp = jnp.maximum((shi - slo) // 16, 1)
        new_lo, new_hi = slo, shi
        for k in range(1, 16):
            m = slo + k * stp
            ok = jnp.sum((kk >= m).astype(jnp.int32)) >= MAX_DETS
            new_lo = jnp.where(ok, jnp.maximum(new_lo, m), new_lo)
            new_hi = jnp.where(ok, new_hi, jnp.minimum(new_hi, m))
        return (new_lo, new_hi)

    slo, _ = lax.fori_loop(0, 7, round16, (lo0, hi0))
    cnt_pass = jnp.sum((kk >= lo0).astype(jnp.int32))
    t = jnp.where(cnt_pass >= MAX_DETS, slo, jnp.int32(-1))

    n_greater = jnp.sum((kk > t).astype(jnp.int32))
    need = (MAX_DETS - n_greater).astype(jnp.float32)
    gt = kk > t
    tie = kk == t
    gcum = _excl_cumsum(gt.astype(jnp.float32))
    tcum = _excl_cumsum(tie.astype(jnp.float32))
    sel = gt | (tie & (tcum < need))
    pos = gcum + jnp.minimum(tcum, need)
    posx = jnp.where(sel, pos, 3.0e4)

    posf = jnp.reshape(posx, (1, AP))
    slots = lax.broadcasted_iota(jnp.int32, (P, 1), 0).astype(jnp.float32)
    oh = (slots == posf).astype(jnp.bfloat16)
    c = _dot11(oh, vb)
    ct = _dot11(vb, oh)

    def pair_of(c3, c2, c1, c0):
        hi = jnp.where(c3 >= 128.0, c3 - 256.0, c3) * 256.0 + c2
        lo = c1 * 256.0 + c0
        return hi, lo

    khp, klp = pair_of(c[:, 9:10], c[:, 8:9], c[:, 7:8], c[:, 6:7])
    khq, klq = pair_of(ct[9:10, :], ct[8:9, :], ct[7:8, :], ct[6:7, :])
    ip = c[:, 10:11] + 256.0 * c[:, 11:12]
    iq = ct[10:11, :] + 256.0 * ct[11:12, :]
    valid_p = lax.broadcasted_iota(jnp.int32, (P, 1), 0).astype(
        jnp.float32) < MAX_DETS
    valid_q = lax.broadcasted_iota(jnp.int32, (1, P), 1) < MAX_DETS
    beats = valid_p & (
        (khp > khq) | ((khp == khq) & ((klp > klq)
                                       | ((klp == klq) & (ip < iq)))))
    rank_q = jnp.sum(beats.astype(jnp.float32), axis=0, keepdims=True)

    out_r = lax.broadcasted_iota(jnp.int32, (P, 1), 0).astype(jnp.float32)
    oh2 = ((out_r == rank_q) & valid_q).astype(jnp.bfloat16)
    d = lax.dot_general(oh2, c[:, 0:6].astype(jnp.bfloat16),
                        (((1,), (0,)), ((), ())),
                        preferred_element_type=jnp.float32)
    det = d[0:MAX_DETS, :]
    score = jnp.maximum(det[:, 4:5], 0.0)
    det = jnp.concatenate([det[:, 0:4], score, det[:, 5:6]], axis=1)
    det_ref[...] = det[None]
    num_ref[...] = jnp.broadcast_to(jnp.minimum(cnt_pass, MAX_DETS),
                                    (1, 1, 128))


@jax.jit
def kernel(p8, p16, p32):
    b = p8.shape[0]
    r8 = p8.reshape(b, 144, 6400)
    r16 = p16.reshape(b, 144, 1600)
    r32 = p32.reshape(b, 144, 400)

    det, num = pl.pallas_call(
        _body,
        grid=(b,),
        in_specs=[
            pl.BlockSpec((1, 144, 6400), lambda i: (i, 0, 0)),
            pl.BlockSpec((1, 144, 1600), lambda i: (i, 0, 0)),
            pl.BlockSpec((1, 144, 400), lambda i: (i, 0, 0)),
        ],
        out_specs=[
            pl.BlockSpec((1, MAX_DETS, 6), lambda i: (i, 0, 0)),
            pl.BlockSpec((1, 1, 128), lambda i: (i, 0, 0)),
        ],
        out_shape=[
            jax.ShapeDtypeStruct((b, MAX_DETS, 6), jnp.float32),
            jax.ShapeDtypeStruct((b, 1, 128), jnp.int32),
        ],
        compiler_params=pltpu.CompilerParams(
            dimension_semantics=("parallel",)),
    )(r8, r16, r32)

    return det, num[:, 0, 0].astype(jnp.int64)
